# baseline (speedup 1.0000x reference)
import functools

import jax
import jax.numpy as jnp
from jax import lax
from jax.experimental import pallas as pl
from jax.experimental.pallas import tpu as pltpu
from jax.experimental.pallas import tpu_sc as plsc

_N_NODE = 1000000
_IDX_CHUNK = 128


@functools.cache
def _make_sc_gather(B, D):
    info = plsc.get_sparse_core_info()
    NC, NS = info.num_cores, info.num_subcores
    NW = NC * NS
    b_per_w = B // NW
    n_chunks = b_per_w // _IDX_CHUNK
    mesh = plsc.VectorSubcoreMesh(core_axis_name="c", subcore_axis_name="s")

    @functools.partial(
        pl.kernel,
        out_type=(
            jax.ShapeDtypeStruct((B, D), jnp.float32),
            jax.ShapeDtypeStruct((B, D), jnp.float32),
        ),
        mesh=mesh,
        scratch_types=[
            pltpu.VMEM((b_per_w,), jnp.int32),
            pltpu.VMEM((b_per_w,), jnp.int32),
            pltpu.VMEM((b_per_w, D), jnp.float32),
            pltpu.VMEM((b_per_w, D), jnp.float32),
            pltpu.SemaphoreType.DMA,
        ],
        compiler_params=pltpu.CompilerParams(use_tc_tiling_on_sc=False),
    )
    def sc_gather(node_hbm, rel_hbm, eidx_hbm, base_hbm, ht_out, rel_out,
                  eidx_v, base_v, ht_v, relrow_v, sem):
        wid = lax.axis_index("s") * NC + lax.axis_index("c")
        off = wid * b_per_w
        pltpu.sync_copy(eidx_hbm.at[pl.ds(off, b_per_w)], eidx_v)
        pltpu.sync_copy(base_hbm.at[pl.ds(off, b_per_w)], base_v)
        for j in range(b_per_w // 16):
            sl = pl.ds(j * 16, 16)
            eidx_v[sl] = eidx_v[sl] - _N_NODE
        copies = []
        for i in range(n_chunks):
            sl = pl.ds(i * _IDX_CHUNK, _IDX_CHUNK)
            copies.append(
                pltpu.async_copy(node_hbm.at[eidx_v.at[sl]], ht_v.at[sl], sem))
            copies.append(
                pltpu.async_copy(rel_hbm.at[base_v.at[sl]], relrow_v.at[sl], sem))
        for cp in copies:
            cp.wait()
        pltpu.sync_copy(ht_v, ht_out.at[pl.ds(off, b_per_w)])
        pltpu.sync_copy(relrow_v, rel_out.at[pl.ds(off, b_per_w)])

    return sc_gather


def _tc_score_body(ht_ref, rel_ref, gt_ref, out_ref):
    a = ht_ref[...]
    b = rel_ref[...]
    aa = jnp.sum(a * a, axis=1, keepdims=True)
    bb = jnp.sum(b * b, axis=1, keepdims=True)
    ab = jnp.sum(a * b, axis=1, keepdims=True)
    eps = jnp.float32(1e-12)
    denom = jnp.maximum(jnp.sqrt(aa), eps) * jnp.maximum(jnp.sqrt(bb), eps)
    x = -(ab / denom) * gt_ref[...]
    sp = jnp.maximum(x, 0.0) + jnp.log1p(jnp.exp(-jnp.abs(x)))
    out_ref[...] = (jnp.sum(sp) * jnp.float32(1.0 / x.shape[0])).reshape(1, 1)


def kernel(node_table, rel_table, base_edge_index, base, ground_truth):
    B = base.shape[0]
    D = node_table.shape[1]
    eidx = base_edge_index.reshape(B)
    ht_rows, rel_rows = _make_sc_gather(B, D)(node_table, rel_table, eidx, base)
    loss = pl.pallas_call(
        _tc_score_body,
        out_shape=jax.ShapeDtypeStruct((1, 1), jnp.float32),
    )(ht_rows, rel_rows, ground_truth)
    return loss[0, 0]
